# baseline (speedup 1.0000x reference)
import jax
import jax.numpy as jnp
from jax.experimental import pallas as pl

BLOCK_ROWS = 512
D = 1024


def _add_body(mod_ref, v_ref, l_ref, ov_ref, ol_ref):
    ov_ref[...] = v_ref[...] + mod_ref[0:1, :]
    ol_ref[...] = l_ref[...] + mod_ref[1:2, :]


def kernel(vision, language, mod_emb):
    b, lv, d = vision.shape
    _, lt, _ = language.shape
    v2 = vision.reshape(b * lv, d)
    l2 = language.reshape(b * lt, d)
    n = b * lv
    grid = (n // BLOCK_ROWS,)

    ov, ol = pl.pallas_call(
        _add_body,
        grid=grid,
        in_specs=[
            pl.BlockSpec((2, d), lambda i: (0, 0)),
            pl.BlockSpec((BLOCK_ROWS, d), lambda i: (i, 0)),
            pl.BlockSpec((BLOCK_ROWS, d), lambda i: (i, 0)),
        ],
        out_specs=[
            pl.BlockSpec((BLOCK_ROWS, d), lambda i: (i, 0)),
            pl.BlockSpec((BLOCK_ROWS, d), lambda i: (i, 0)),
        ],
        out_shape=[
            jax.ShapeDtypeStruct((n, d), jnp.float32),
            jax.ShapeDtypeStruct((n, d), jnp.float32),
        ],
    )(mod_emb, v2, l2)

    return ov.reshape(b, lv, d), ol.reshape(b, lt, d)


# speedup vs baseline: 1.0113x; 1.0113x over previous
import jax
import jax.numpy as jnp
from jax.experimental import pallas as pl

BLOCK_ROWS = 1024
D = 1024


def _add_body(mod_ref, v_ref, l_ref, ov_ref, ol_ref):
    ov_ref[...] = v_ref[...] + mod_ref[0:1, :]
    ol_ref[...] = l_ref[...] + mod_ref[1:2, :]


def kernel(vision, language, mod_emb):
    b, lv, d = vision.shape
    _, lt, _ = language.shape
    v2 = vision.reshape(b * lv, d)
    l2 = language.reshape(b * lt, d)
    n = b * lv
    grid = (n // BLOCK_ROWS,)

    ov, ol = pl.pallas_call(
        _add_body,
        grid=grid,
        in_specs=[
            pl.BlockSpec((2, d), lambda i: (0, 0)),
            pl.BlockSpec((BLOCK_ROWS, d), lambda i: (i, 0)),
            pl.BlockSpec((BLOCK_ROWS, d), lambda i: (i, 0)),
        ],
        out_specs=[
            pl.BlockSpec((BLOCK_ROWS, d), lambda i: (i, 0)),
            pl.BlockSpec((BLOCK_ROWS, d), lambda i: (i, 0)),
        ],
        out_shape=[
            jax.ShapeDtypeStruct((n, d), jnp.float32),
            jax.ShapeDtypeStruct((n, d), jnp.float32),
        ],
    )(mod_emb, v2, l2)

    return ov.reshape(b, lv, d), ol.reshape(b, lt, d)
